# baseline (speedup 1.0000x reference)
import functools

import jax
import jax.numpy as jnp
from jax.experimental import pallas as pl
from jax.experimental.pallas import tpu as pltpu

N_EXPERTS = 8
TOP_K = 2
N_EMBED = 1024
EXPERT_DIM = 512
T_BLOCK = 256


def _routing(logits):
    m = jnp.max(logits, axis=-1, keepdims=True)
    p = jnp.exp(logits - m)
    s = p / jnp.sum(p, axis=-1, keepdims=True)
    lane = jax.lax.broadcasted_iota(jnp.int32, s.shape, 1)
    v1 = jnp.max(s, axis=-1, keepdims=True)
    i1 = jnp.min(jnp.where(s == v1, lane, N_EXPERTS), axis=-1, keepdims=True)
    s2 = jnp.where(lane == i1, -jnp.inf, s)
    v2 = jnp.max(s2, axis=-1, keepdims=True)
    i2 = jnp.min(jnp.where(s2 == v2, lane, N_EXPERTS), axis=-1, keepdims=True)
    denom = v1 + v2 + 1e-9
    w1 = v1 / denom
    w2 = v2 / denom
    d2 = w1 + w2 + 1e-9
    r1 = w1 / d2
    r2 = w2 / d2
    routed = jnp.where(lane == i1, r1, 0.0) + jnp.where(lane == i2, r2, 0.0)
    return routed


def _moe_body(x_ref, gw_ref, gup_ref, dp_ref, out_ref, routed_s):
    e = pl.program_id(1)

    @pl.when(e == 0)
    def _():
        logits = jnp.dot(x_ref[...], gw_ref[...].T,
                         preferred_element_type=jnp.float32)
        routed_s[...] = _routing(logits)
        out_ref[...] = jnp.zeros_like(out_ref)

    x = x_ref[...]
    gu = jnp.dot(x, gup_ref[0], preferred_element_type=jnp.float32)
    gate = gu[:, :EXPERT_DIM]
    up = gu[:, EXPERT_DIM:]
    act = (gate * jax.nn.sigmoid(gate)) * up
    eo = jnp.dot(act, dp_ref[0], preferred_element_type=jnp.float32)
    routed = routed_s[...]
    lane = jax.lax.broadcasted_iota(jnp.int32, routed.shape, 1)
    w = jnp.sum(jnp.where(lane == e, routed, 0.0), axis=1, keepdims=True)
    out_ref[...] += w * eo


@jax.jit
def kernel(x, gate_w, gate_up_proj, down_proj):
    Bb, Tt, H = x.shape
    hidden = x.reshape(Tt, H)
    n_tb = Tt // T_BLOCK
    out = pl.pallas_call(
        _moe_body,
        grid=(n_tb, N_EXPERTS),
        in_specs=[
            pl.BlockSpec((T_BLOCK, H), lambda t, e: (t, 0)),
            pl.BlockSpec((N_EXPERTS, H), lambda t, e: (0, 0)),
            pl.BlockSpec((1, H, 2 * EXPERT_DIM), lambda t, e: (e, 0, 0)),
            pl.BlockSpec((1, EXPERT_DIM, H), lambda t, e: (e, 0, 0)),
        ],
        out_specs=pl.BlockSpec((T_BLOCK, H), lambda t, e: (t, 0)),
        out_shape=jax.ShapeDtypeStruct((Tt, H), jnp.float32),
        scratch_shapes=[pltpu.VMEM((T_BLOCK, N_EXPERTS), jnp.float32)],
    )(hidden, gate_w, gate_up_proj, down_proj)
    return out.reshape(Bb, Tt, H)
